# baseline (speedup 1.0000x reference)
import functools

import jax
import jax.numpy as jnp
from jax import lax
from jax.experimental import pallas as pl
from jax.experimental.pallas import tpu as pltpu
from jax.experimental.pallas import tpu_sc as plsc

_NC = 2
_NS = 16
_NW = _NC * _NS

_SEED = 16
_BLOCK = _SEED * _NS


def _make_broadcast_kernel(S, D, dtype):
    b_per_w = S // _NW
    n_dma = b_per_w // _BLOCK
    mesh = plsc.VectorSubcoreMesh(core_axis_name="c", subcore_axis_name="s")

    @functools.partial(
        pl.kernel,
        out_type=jax.ShapeDtypeStruct((S, D), dtype),
        mesh=mesh,
        scratch_types=[
            pltpu.VMEM((_SEED,), jnp.int32),
            pltpu.VMEM((_SEED, D), dtype),
            pltpu.VMEM_SHARED((_BLOCK, D), dtype),
            pltpu.SemaphoreType.DMA,
            pltpu.SemaphoreType.DMA,
        ],
    )
    def broadcast_kernel(table_hbm, idx_hbm, out_hbm, idx_v, row_v, shared_v,
                         gsem, wsem):
        cid = lax.axis_index("c")
        sid = lax.axis_index("s")
        base = (cid * _NS + sid) * b_per_w

        @pl.when(sid == 0)
        def _():
            pltpu.sync_copy(idx_hbm, idx_v)
            pltpu.async_copy(table_hbm.at[idx_v], row_v, gsem).wait()
            pltpu.sync_copy(row_v, shared_v.at[pl.ds(0, _SEED)])

        plsc.subcore_barrier()

        @pl.when(sid > 0)
        def _():
            pltpu.sync_copy(shared_v.at[pl.ds(0, _SEED)], row_v)
            pltpu.sync_copy(row_v, shared_v.at[pl.ds(sid * _SEED, _SEED)])

        plsc.subcore_barrier()

        mine = shared_v.at[pl.ds(sid * _SEED, _SEED)]
        copies = [
            pltpu.async_copy(
                mine, out_hbm.at[pl.ds(base + j * _SEED, _SEED)], wsem
            )
            for j in range(b_per_w // _SEED)
        ]
        for c in copies:
            c.wait()

    return broadcast_kernel


def kernel(embeddings, modality_embedding, token_type_id):
    S = embeddings.shape[1]
    D = modality_embedding.shape[1]
    idx = jnp.full((_SEED,), token_type_id, dtype=jnp.int32)
    fn = _make_broadcast_kernel(S, D, modality_embedding.dtype)
    return fn(modality_embedding, idx)


# speedup vs baseline: 1.1768x; 1.1768x over previous
import functools

import jax
import jax.numpy as jnp
from jax import lax
from jax.experimental import pallas as pl
from jax.experimental.pallas import tpu as pltpu
from jax.experimental.pallas import tpu_sc as plsc

_NC = 2
_NS = 16
_NW = _NC * _NS

_SEED = 16
_BLOCK = _SEED * _NS


def _make_broadcast_kernel(S, D, dtype):
    b_per_w = S // _NW
    n_dma = b_per_w // _BLOCK
    mesh = plsc.VectorSubcoreMesh(core_axis_name="c", subcore_axis_name="s")

    @functools.partial(
        pl.kernel,
        out_type=jax.ShapeDtypeStruct((S, D), dtype),
        mesh=mesh,
        scratch_types=[
            pltpu.VMEM((_SEED,), jnp.int32),
            pltpu.VMEM((_SEED, D), dtype),
            pltpu.VMEM_SHARED((_BLOCK, D), dtype),
            pltpu.SemaphoreType.DMA,
            pltpu.SemaphoreType.DMA,
        ],
    )
    def broadcast_kernel(table_hbm, idx_hbm, out_hbm, idx_v, row_v, shared_v,
                         gsem, wsem):
        cid = lax.axis_index("c")
        sid = lax.axis_index("s")
        base = (cid * _NS + sid) * b_per_w

        @pl.when(sid == 0)
        def _():
            pltpu.sync_copy(idx_hbm, idx_v)
            pltpu.async_copy(table_hbm.at[idx_v], row_v, gsem).wait()
            pltpu.sync_copy(row_v, shared_v.at[pl.ds(0, _SEED)])

        plsc.subcore_barrier()

        @pl.when(sid > 0)
        def _():
            pltpu.sync_copy(shared_v.at[pl.ds(0, _SEED)], row_v)
            pltpu.sync_copy(row_v, shared_v.at[pl.ds(sid * _SEED, _SEED)])

        plsc.subcore_barrier()

        mine = shared_v.at[pl.ds(sid * _SEED, _SEED)]
        n_chunks = b_per_w // _SEED
        n_spmem = (n_chunks * 9) // 16
        copies = [
            pltpu.async_copy(
                mine if j < n_spmem else row_v,
                out_hbm.at[pl.ds(base + j * _SEED, _SEED)],
                wsem if j < n_spmem else gsem,
            )
            for j in range(n_chunks)
        ]
        for c in copies:
            c.wait()

    return broadcast_kernel


def kernel(embeddings, modality_embedding, token_type_id):
    S = embeddings.shape[1]
    D = modality_embedding.shape[1]
    idx = jnp.full((_SEED,), token_type_id, dtype=jnp.int32)
    fn = _make_broadcast_kernel(S, D, modality_embedding.dtype)
    return fn(modality_embedding, idx)


# speedup vs baseline: 1.1769x; 1.0001x over previous
import functools

import jax
import jax.numpy as jnp
from jax import lax
from jax.experimental import pallas as pl
from jax.experimental.pallas import tpu as pltpu
from jax.experimental.pallas import tpu_sc as plsc

_NC = 2
_NS = 16
_NW = _NC * _NS

_SEED = 16
_BLOCK = _SEED * _NS


def _make_broadcast_kernel(S, D, dtype):
    b_per_w = S // _NW
    n_dma = b_per_w // _BLOCK
    mesh = plsc.VectorSubcoreMesh(core_axis_name="c", subcore_axis_name="s")

    @functools.partial(
        pl.kernel,
        out_type=jax.ShapeDtypeStruct((S, D), dtype),
        mesh=mesh,
        scratch_types=[
            pltpu.VMEM((_SEED,), jnp.int32),
            pltpu.VMEM((_SEED, D), dtype),
            pltpu.VMEM_SHARED((_BLOCK, D), dtype),
            pltpu.SemaphoreType.DMA,
            pltpu.SemaphoreType.DMA,
        ],
    )
    def broadcast_kernel(table_hbm, idx_hbm, out_hbm, idx_v, row_v, shared_v,
                         gsem, wsem):
        cid = lax.axis_index("c")
        sid = lax.axis_index("s")
        base = (cid * _NS + sid) * b_per_w

        @pl.when(sid == 0)
        def _():
            pltpu.sync_copy(idx_hbm, idx_v)
            pltpu.async_copy(table_hbm.at[idx_v], row_v, gsem).wait()
            pltpu.sync_copy(row_v, shared_v.at[pl.ds(0, _SEED)])

        plsc.subcore_barrier()

        @pl.when(sid > 0)
        def _():
            pltpu.sync_copy(shared_v.at[pl.ds(0, _SEED)], row_v)
            pltpu.sync_copy(row_v, shared_v.at[pl.ds(sid * _SEED, _SEED)])

        plsc.subcore_barrier()

        mine = shared_v.at[pl.ds(sid * _SEED, _SEED)]
        n_chunks = b_per_w // _SEED
        spmem_share = [(j * 9) // 16 != ((j + 1) * 9) // 16
                       for j in range(n_chunks)]
        copies = [
            pltpu.async_copy(
                mine if use_spmem else row_v,
                out_hbm.at[pl.ds(base + j * _SEED, _SEED)],
                wsem if use_spmem else gsem,
            )
            for j, use_spmem in enumerate(spmem_share)
        ]
        for c in copies:
            c.wait()

    return broadcast_kernel


def kernel(embeddings, modality_embedding, token_type_id):
    S = embeddings.shape[1]
    D = modality_embedding.shape[1]
    idx = jnp.full((_SEED,), token_type_id, dtype=jnp.int32)
    fn = _make_broadcast_kernel(S, D, modality_embedding.dtype)
    return fn(modality_embedding, idx)


# speedup vs baseline: 1.2086x; 1.0269x over previous
import functools

import jax
import jax.numpy as jnp
from jax import lax
from jax.experimental import pallas as pl
from jax.experimental.pallas import tpu as pltpu
from jax.experimental.pallas import tpu_sc as plsc

_NC = 2
_NS = 16
_NW = _NC * _NS

_SEED = 16
_BLOCK = _SEED * _NS


def _make_broadcast_kernel(S, D, dtype):
    b_per_w = S // _NW
    n_dma = b_per_w // _BLOCK
    mesh = plsc.VectorSubcoreMesh(core_axis_name="c", subcore_axis_name="s")

    @functools.partial(
        pl.kernel,
        out_type=jax.ShapeDtypeStruct((S, D), dtype),
        mesh=mesh,
        scratch_types=[
            pltpu.VMEM((_SEED,), jnp.int32),
            pltpu.VMEM((_SEED, D), dtype),
            pltpu.VMEM_SHARED((_BLOCK, D), dtype),
            pltpu.SemaphoreType.DMA,
            pltpu.SemaphoreType.DMA,
        ],
    )
    def broadcast_kernel(table_hbm, idx_hbm, out_hbm, idx_v, row_v, shared_v,
                         gsem, wsem):
        cid = lax.axis_index("c")
        sid = lax.axis_index("s")
        base = (cid * _NS + sid) * b_per_w

        @pl.when(sid == 0)
        def _():
            pltpu.sync_copy(idx_hbm, idx_v)
            pltpu.async_copy(table_hbm.at[idx_v], row_v, gsem).wait()
            pltpu.sync_copy(row_v, shared_v.at[pl.ds(0, _SEED)])

        plsc.subcore_barrier()

        seed = shared_v.at[pl.ds(0, _SEED)]
        n_chunks = b_per_w // _SEED
        spmem_share = [(j * 9) // 16 != ((j + 1) * 9) // 16
                       for j in range(n_chunks)]
        copies = [
            pltpu.async_copy(
                seed, out_hbm.at[pl.ds(base + j * _SEED, _SEED)], wsem
            )
            for j, use_spmem in enumerate(spmem_share) if use_spmem
        ]
        @pl.when(sid > 0)
        def _():
            pltpu.sync_copy(seed, row_v)
        copies += [
            pltpu.async_copy(
                row_v, out_hbm.at[pl.ds(base + j * _SEED, _SEED)], gsem
            )
            for j, use_spmem in enumerate(spmem_share) if not use_spmem
        ]
        for c in copies:
            c.wait()

    return broadcast_kernel


def kernel(embeddings, modality_embedding, token_type_id):
    S = embeddings.shape[1]
    D = modality_embedding.shape[1]
    idx = jnp.full((_SEED,), token_type_id, dtype=jnp.int32)
    fn = _make_broadcast_kernel(S, D, modality_embedding.dtype)
    return fn(modality_embedding, idx)
